# baseline (speedup 1.0000x reference)
import functools

import jax
import jax.numpy as jnp
from jax import lax
from jax.experimental import pallas as pl
from jax.experimental.pallas import tpu as pltpu
from jax.experimental.pallas import tpu_sc as plsc

D = 128
LANES = 16
NC = 2
NS = 16
NW = NC * NS
CHUNK = 128
NBE = 4
H_SUB = D // LANES


def _sc_spmm(edges, embeds, n_chunks, n_real):
    rows_per_tile = -(-n_real // (NS * 8)) * 8
    n = rows_per_tile * NS

    mesh = plsc.VectorSubcoreMesh(core_axis_name="c", subcore_axis_name="s")

    @functools.partial(
        pl.kernel,
        mesh=mesh,
        out_type=jax.ShapeDtypeStruct((NC, n, D), jnp.float32),
        scratch_types=[
            pltpu.VMEM((NBE, 3, CHUNK), jnp.int32),
            pltpu.VMEM((2, CHUNK, D), jnp.float32),
            pltpu.VMEM_SHARED((n, D), jnp.float32),
            pltpu.SemaphoreType.DMA((NBE,)),
            pltpu.SemaphoreType.DMA((2,)),
        ],
    )
    def k(edges_hbm, embeds_hbm, out_hbm, ibuf, gbuf, accum, esem, gsem):
        c = lax.axis_index("c")
        s = lax.axis_index("s")
        wid = c * NS + s

        zrow = jnp.zeros((LANES,), jnp.float32)
        for i in range(CHUNK):
            for h in range(H_SUB):
                gbuf[0, i, pl.ds(h * LANES, LANES)] = zrow

        r0 = s * rows_per_tile
        full, rem = divmod(rows_per_tile, CHUNK)
        for b in range(full):
            pltpu.sync_copy(gbuf.at[0],
                            accum.at[pl.ds(r0 + b * CHUNK, CHUNK)])
        if rem:
            pltpu.sync_copy(gbuf.at[0, pl.ds(0, rem)],
                            accum.at[pl.ds(r0 + full * CHUNK, rem)])
        plsc.subcore_barrier()

        def edge_descs(t, be):
            return (
                pltpu.make_async_copy(
                    edges_hbm.at[wid, t], ibuf.at[be], esem.at[be]),
            )

        def gather_desc(be, pb):
            return pltpu.make_async_copy(
                embeds_hbm.at[ibuf.at[be, 0]], gbuf.at[pb], gsem.at[pb])

        for t0 in range(3):
            for d_ in edge_descs(t0, t0 % NBE):
                d_.start()
        for d_ in edge_descs(0, 0):
            d_.wait()
        gather_desc(0, 0).start()

        def chunk_body(t, carry):
            for d_ in edge_descs(t + 3, lax.rem(t + 3, NBE)):
                d_.start()
            be1 = lax.rem(t + 1, NBE)
            for d_ in edge_descs(t + 1, be1):
                d_.wait()
            pb1 = lax.rem(t + 1, 2)
            gather_desc(be1, pb1).start()

            be = lax.rem(t, NBE)
            pb = lax.rem(t, 2)
            gather_desc(be, pb).wait()

            for g in range(CHUNK // LANES):
                base = g * LANES
                v16 = lax.bitcast_convert_type(
                    ibuf[be, 2, pl.ds(base, LANES)], jnp.float32)
                for l in range(LANES):
                    vb = jnp.full((LANES,), v16[l], dtype=jnp.float32)
                    e = base + l
                    for h in range(H_SUB):
                        sl = pl.ds(h * LANES, LANES)
                        gbuf[pb, e, sl] = gbuf[pb, e, sl] * vb

            pltpu.sync_copy(gbuf.at[pb], accum.at[ibuf.at[be, 1]],
                            add=True)
            return carry
        lax.fori_loop(0, n_chunks, chunk_body, 0)

        gather_desc(lax.rem(n_chunks, NBE), lax.rem(n_chunks, 2)).wait()
        for td in (n_chunks + 1, n_chunks + 2):
            for d_ in edge_descs(td, td % NBE):
                d_.wait()

        plsc.subcore_barrier()
        pltpu.sync_copy(accum.at[pl.ds(r0, rows_per_tile)],
                        out_hbm.at[c, pl.ds(r0, rows_per_tile)])

    return k(edges, embeds)


def _combine_body(p_ref, o_ref):
    o_ref[...] = p_ref[0] + p_ref[1]


def _combine(partials, n):
    d = partials.shape[2]
    blk = 2000
    return pl.pallas_call(
        _combine_body,
        grid=(n // blk,),
        in_specs=[pl.BlockSpec((NC, blk, d), lambda i: (0, i, 0))],
        out_specs=pl.BlockSpec((blk, d), lambda i: (i, 0)),
        out_shape=jax.ShapeDtypeStruct((n, d), jnp.float32),
    )(partials)


@jax.jit
def kernel(edge_index, values, embeds):
    n = embeds.shape[0]
    e = values.shape[0]
    rows = edge_index[0].astype(jnp.int32)
    cols = edge_index[1].astype(jnp.int32)
    vals = values.astype(jnp.float32)

    per_tile = NW * CHUNK
    n_chunks = -(-e // per_tile)
    e_pad = n_chunks * per_tile
    pad = e_pad - e
    if pad:
        pad_idx = (jnp.arange(pad, dtype=jnp.int32) * 17) % n
        rows = jnp.concatenate([rows, pad_idx])
        cols = jnp.concatenate([cols, pad_idx])
        vals = jnp.concatenate([vals, jnp.zeros((pad,), jnp.float32)])

    cols = cols.reshape(NW, n_chunks, 1, CHUNK)
    rows = rows.reshape(NW, n_chunks, 1, CHUNK)
    vals_i = lax.bitcast_convert_type(vals, jnp.int32)
    vals_i = vals_i.reshape(NW, n_chunks, 1, CHUNK)
    edges = jnp.concatenate([cols, rows, vals_i], axis=2)
    edges = jnp.pad(edges, ((0, 0), (0, 3), (0, 0), (0, 0)))

    partials = _sc_spmm(edges, embeds, n_chunks, n)
    return _combine(partials, n)


# speedup vs baseline: 1.2195x; 1.2195x over previous
import functools

import jax
import jax.numpy as jnp
from jax import lax
from jax.experimental import pallas as pl
from jax.experimental.pallas import tpu as pltpu
from jax.experimental.pallas import tpu_sc as plsc

D = 128
LANES = 16
NC = 2
NS = 16
NW = NC * NS
CHUNK = 128
NBE = 4
H_SUB = D // LANES


def _sc_spmm(edges, embeds, n_chunks, n_real):
    rows_per_tile = -(-n_real // (NS * 8)) * 8
    n = rows_per_tile * NS

    mesh = plsc.VectorSubcoreMesh(core_axis_name="c", subcore_axis_name="s")

    @functools.partial(
        pl.kernel,
        mesh=mesh,
        out_type=jax.ShapeDtypeStruct((NC, n, D), jnp.float32),
        scratch_types=[
            pltpu.VMEM((NBE, 3, CHUNK), jnp.int32),
            pltpu.VMEM((CHUNK, D), jnp.float32),
            pltpu.VMEM_SHARED((n, D), jnp.float32),
            pltpu.SemaphoreType.DMA((NBE,)),
            pltpu.SemaphoreType.DMA,
        ],
    )
    def k(edges_hbm, embeds_hbm, out_hbm, ibuf, gbuf, accum, esem, gsem):
        c = lax.axis_index("c")
        s = lax.axis_index("s")
        wid = c * NS + s

        zrow = jnp.zeros((LANES,), jnp.float32)
        for i in range(CHUNK):
            for h in range(H_SUB):
                gbuf[i, pl.ds(h * LANES, LANES)] = zrow

        r0 = s * rows_per_tile
        full, rem = divmod(rows_per_tile, CHUNK)
        for b in range(full):
            pltpu.sync_copy(gbuf, accum.at[pl.ds(r0 + b * CHUNK, CHUNK)])
        if rem:
            pltpu.sync_copy(gbuf.at[pl.ds(0, rem)],
                            accum.at[pl.ds(r0 + full * CHUNK, rem)])
        plsc.subcore_barrier()

        def edge_descs(t, be):
            return (
                pltpu.make_async_copy(
                    edges_hbm.at[wid, t], ibuf.at[be], esem.at[be]),
            )

        def gather_desc(be):
            return pltpu.make_async_copy(
                embeds_hbm.at[ibuf.at[be, 0]], gbuf, gsem)

        for t0 in range(2):
            for d_ in edge_descs(t0, t0 % NBE):
                d_.start()

        def chunk_body(t, carry):
            for d_ in edge_descs(t + 2, lax.rem(t + 2, NBE)):
                d_.start()
            be = lax.rem(t, NBE)
            for d_ in edge_descs(t, be):
                d_.wait()

            gather_desc(be).start()
            gather_desc(be).wait()

            for g in range(CHUNK // LANES):
                base = g * LANES
                v16 = lax.bitcast_convert_type(
                    ibuf[be, 2, pl.ds(base, LANES)], jnp.float32)
                for l in range(LANES):
                    vb = jnp.full((LANES,), v16[l], dtype=jnp.float32)
                    e = base + l
                    for h in range(H_SUB):
                        sl = pl.ds(h * LANES, LANES)
                        gbuf[e, sl] = gbuf[e, sl] * vb

            pltpu.sync_copy(gbuf, accum.at[ibuf.at[be, 1]], add=True)
            return carry
        lax.fori_loop(0, n_chunks, chunk_body, 0)

        for td in (n_chunks, n_chunks + 1):
            for d_ in edge_descs(td, td % NBE):
                d_.wait()

        plsc.subcore_barrier()
        pltpu.sync_copy(accum.at[pl.ds(r0, rows_per_tile)],
                        out_hbm.at[c, pl.ds(r0, rows_per_tile)])

    return k(edges, embeds)


def _combine_body(p_ref, o_ref):
    o_ref[...] = p_ref[0] + p_ref[1]


def _combine(partials, n):
    d = partials.shape[2]
    blk = 2000
    return pl.pallas_call(
        _combine_body,
        grid=(n // blk,),
        in_specs=[pl.BlockSpec((NC, blk, d), lambda i: (0, i, 0))],
        out_specs=pl.BlockSpec((blk, d), lambda i: (i, 0)),
        out_shape=jax.ShapeDtypeStruct((n, d), jnp.float32),
    )(partials)


@jax.jit
def kernel(edge_index, values, embeds):
    n = embeds.shape[0]
    e = values.shape[0]
    rows = edge_index[0].astype(jnp.int32)
    cols = edge_index[1].astype(jnp.int32)
    vals = values.astype(jnp.float32)

    per_tile = NW * CHUNK
    n_chunks = -(-e // per_tile)
    e_pad = n_chunks * per_tile
    pad = e_pad - e
    if pad:
        pad_idx = (jnp.arange(pad, dtype=jnp.int32) * 17) % n
        rows = jnp.concatenate([rows, pad_idx])
        cols = jnp.concatenate([cols, pad_idx])
        vals = jnp.concatenate([vals, jnp.zeros((pad,), jnp.float32)])

    cols = cols.reshape(NW, n_chunks, 1, CHUNK)
    rows = rows.reshape(NW, n_chunks, 1, CHUNK)
    vals_i = lax.bitcast_convert_type(vals, jnp.int32)
    vals_i = vals_i.reshape(NW, n_chunks, 1, CHUNK)
    edges = jnp.concatenate([cols, rows, vals_i], axis=2)
    edges = jnp.pad(edges, ((0, 0), (0, 2), (0, 0), (0, 0)))

    partials = _sc_spmm(edges, embeds, n_chunks, n)
    return _combine(partials, n)
